# baseline (speedup 1.0000x reference)
import functools

import jax
import jax.numpy as jnp
import numpy as np
from jax.experimental import pallas as pl
from jax.experimental.pallas import tpu as pltpu

BATCH = 1
SEQ = 2048
DIM = 768
HEADS = 12
DIM_HEAD = 64
SLIDING = 64
CBS = 16
SBS = 16
NUM_SEL = 4
NUM_MEM = 4
SCALE = DIM_HEAD ** -0.5
NBLK = SEQ // CBS
CTX = NUM_MEM + NBLK
NEG = -1e30

BQ = 256
BK = 256
NT = SEQ // BK
GQ = SEQ // BQ


def _rope_tables():
    inv = 1.0 / (10000.0 ** (np.arange(0, DIM_HEAD, 2, dtype=np.float64) / DIM_HEAD))
    f = np.arange(SEQ, dtype=np.float64)[:, None] * inv[None, :]
    f = np.repeat(f, 2, axis=-1)
    cos = np.cos(f.astype(np.float32)).astype(np.float32)
    sin = np.sin(f.astype(np.float32)).astype(np.float32)
    P = np.zeros((DIM_HEAD, DIM_HEAD), np.float32)
    for k in range(DIM_HEAD // 2):
        P[2 * k + 1, 2 * k] = -1.0
        P[2 * k, 2 * k + 1] = 1.0
    E16 = np.zeros((BK // SBS, BK), np.float32)
    for b in range(BK // SBS):
        E16[b, b * SBS:(b + 1) * SBS] = 1.0
    return jnp.asarray(cos), jnp.asarray(sin), jnp.asarray(P), jnp.asarray(E16)


def _gate_selectors():
    sels = []
    for j in range(3):
        G = np.zeros((3 * HEADS, DIM), np.float32)
        for h in range(HEADS):
            G[h * 3 + j, h * DIM_HEAD:(h + 1) * DIM_HEAD] = 1.0
        sels.append(jnp.asarray(G))
    return sels


def _qkv_kernel(x_ref, gn_ref, wqkv_ref, wcomb_ref, qkv_ref, gate_ref):
    x = x_ref[...]
    ms = jnp.mean(x * x, axis=-1, keepdims=True)
    xn = x * jax.lax.rsqrt(ms + jnp.finfo(jnp.float32).eps) * gn_ref[...]
    qkv_ref[...] = jnp.dot(xn, wqkv_ref[...], preferred_element_type=jnp.float32)
    gate_ref[...] = jnp.dot(xn, wcomb_ref[...], preferred_element_type=jnp.float32)


def _cmlp_kernel(kc_ref, vc_ref, kin_ref, vin_ref, wk1_ref, bk1_ref, wk2_ref,
                 bk2_ref, wv1_ref, bv1_ref, wv2_ref, bv2_ref, ck_ref, cv_ref):
    kc = kc_ref[...] + kin_ref[...]
    vc = vc_ref[...] + vin_ref[...]
    h1 = jnp.maximum(jnp.dot(kc, wk1_ref[...], preferred_element_type=jnp.float32) + bk1_ref[...], 0.0)
    ck_ref[...] = jnp.dot(h1, wk2_ref[...], preferred_element_type=jnp.float32) + bk2_ref[...]
    h2 = jnp.maximum(jnp.dot(vc, wv1_ref[...], preferred_element_type=jnp.float32) + bv1_ref[...], 0.0)
    cv_ref[...] = jnp.dot(h2, wv2_ref[...], preferred_element_type=jnp.float32) + bv2_ref[...]


def _cattn_kernel(q_ref, ck_ref, cv_ref, co_ref, sidx_ref, sval_ref):
    g = pl.program_id(1)
    q = q_ref[0]
    ck = ck_ref[0]
    cv = cv_ref[0]
    sim = jax.lax.dot_general(q, ck, (((1,), (1,)), ((), ())),
                              preferred_element_type=jnp.float32) * SCALE
    row = g * BQ + jax.lax.broadcasted_iota(jnp.int32, (BQ, CTX), 0)
    col = jax.lax.broadcasted_iota(jnp.int32, (BQ, CTX), 1)
    ckseq = jnp.where(col < NUM_MEM, -1, (col - NUM_MEM + 1) * CBS - 1)
    sim = jnp.where(ckseq < row, sim, NEG)
    m = jnp.max(sim, axis=-1, keepdims=True)
    e = jnp.exp(sim - m)
    p = e / jnp.sum(e, axis=-1, keepdims=True)
    co_ref[0] = jnp.dot(p, cv, preferred_element_type=jnp.float32)
    work = jnp.where(col >= NUM_MEM, p, -1.0)
    idxs, vals = [], []
    for _ in range(NUM_SEL):
        mval = jnp.max(work, axis=-1, keepdims=True)
        cand = jnp.where(work == mval, col, jnp.int32(1 << 30))
        midx = jnp.min(cand, axis=-1, keepdims=True)
        vals.append(mval)
        idxs.append(midx - NUM_MEM)
        work = jnp.where(col == midx, -1.0, work)
    sidx_ref[0] = jnp.concatenate(idxs, axis=-1)
    sval_ref[0] = jnp.concatenate(vals, axis=-1)


NB_T = BK // SBS


def _fs_kernel(q_ref, k_ref, v_ref, cos_ref, sin_ref, p64_ref, e16_ref,
               sidx_ref, sval_ref, fo_ref, lo_ref, rk_ref, vext_ref, mk_ref):
    g = pl.program_id(1)
    p64 = p64_ref[...]

    @pl.when(g == 0)
    def _():
        kk = k_ref[0]
        rk = kk * cos_ref[...] + jnp.dot(
            kk, p64, preferred_element_type=jnp.float32) * sin_ref[...]
        rk_ref[...] = rk
        vv = v_ref[0]
        vext_ref[:, :DIM_HEAD] = vv
        lane = jax.lax.broadcasted_iota(jnp.int32, (SEQ, DIM_HEAD), 1)
        vext_ref[:, DIM_HEAD:] = jnp.where(lane == 0, 1.0, 0.0)
        mk_ref[...] = jnp.max(
            jnp.sum(rk * rk, axis=-1, keepdims=True), axis=0, keepdims=True)

    qb = q_ref[0]
    cosq = cos_ref[pl.ds(g * BQ, BQ), :]
    sinq = sin_ref[pl.ds(g * BQ, BQ), :]
    rq = (qb * cosq + jnp.dot(qb, p64, preferred_element_type=jnp.float32)
          * sinq) * SCALE
    nq = jnp.sqrt(jnp.sum(rq * rq, axis=-1, keepdims=True))
    m0 = nq * jnp.sqrt(mk_ref[...])

    qpos_r = g * BQ + jax.lax.broadcasted_iota(jnp.int32, (BQ, 1), 0)
    own_w = qpos_r // SBS
    sidx = sidx_ref[0]
    valid = sval_ref[0] > 1e-10

    e16 = e16_ref[...]
    colb = jax.lax.broadcasted_iota(jnp.int32, (BQ, NB_T), 1)

    def tile_body(t, acc):
        kt = rk_ref[pl.ds(t * BK, BK), :]
        vt = vext_ref[pl.ds(t * BK, BK), :]
        s = jax.lax.dot_general(rq, kt, (((1,), (1,)), ((), ())),
                                preferred_element_type=jnp.float32)
        wb = jnp.zeros((BQ, NB_T), jnp.float32)
        jbb = t * NB_T + colb
        for si in range(NUM_SEL):
            wb += ((sidx[:, si:si + 1] == jbb) & valid[:, si:si + 1]).astype(jnp.float32)
        w = jnp.dot(wb, e16, preferred_element_type=jnp.float32)
        pt = w * jnp.exp(s - m0)
        return acc + jnp.dot(pt, vt, preferred_element_type=jnp.float32)

    acc = jax.lax.fori_loop(0, g + 1, tile_body,
                            jnp.zeros((BQ, 2 * DIM_HEAD), jnp.float32))

    SW = BQ + 2 * SLIDING
    start = jnp.maximum(g * BQ - 2 * SLIDING, 0)
    ks = rk_ref[pl.ds(start, SW), :]
    vs = vext_ref[pl.ds(start, SW), :]
    bsim = jax.lax.dot_general(rq, ks, (((1,), (1,)), ((), ())),
                               preferred_element_type=jnp.float32)
    kpos2 = start + jax.lax.broadcasted_iota(jnp.int32, (BQ, SW), 1)
    qpos2 = g * BQ + jax.lax.broadcasted_iota(jnp.int32, (BQ, SW), 0)
    causal = kpos2 <= qpos2
    eb = jnp.exp(bsim - m0)
    e_sl = jnp.where(causal & (qpos2 - kpos2 <= SLIDING), eb, 0.0)
    sl_ext = jnp.dot(e_sl, vs, preferred_element_type=jnp.float32)
    lo_ref[0] = sl_ext[:, :DIM_HEAD] / sl_ext[:, DIM_HEAD:DIM_HEAD + 1]
    e_own = jnp.where(causal & ((kpos2 // SBS) == own_w), eb, 0.0)
    acc = acc + jnp.dot(e_own, vs, preferred_element_type=jnp.float32)
    fo_ref[0] = acc[:, :DIM_HEAD] / acc[:, DIM_HEAD:DIM_HEAD + 1]


def _comb_kernel(gate_ref, bcomb_ref, co_ref, fo_ref, lo_ref, g0_ref, g1_ref,
                 g2_ref, wout_ref, out_ref):
    sg = jax.nn.sigmoid(gate_ref[...] + bcomb_ref[...])
    o = (jnp.dot(sg, g0_ref[...], preferred_element_type=jnp.float32) * co_ref[...]
         + jnp.dot(sg, g1_ref[...], preferred_element_type=jnp.float32) * fo_ref[...]
         + jnp.dot(sg, g2_ref[...], preferred_element_type=jnp.float32) * lo_ref[...])
    out_ref[...] = jnp.dot(o, wout_ref[...], preferred_element_type=jnp.float32)


def kernel(inp, g_norm, W_qkv, mem_kv, k_intra, v_intra, Wk1, bk1, Wk2, bk2,
           Wv1, bv1, Wv2, bv2, W_comb, b_comb, W_out):
    n, h, dh = SEQ, HEADS, DIM_HEAD
    inner = h * dh
    cdim = CBS * dh
    x2 = inp.reshape(n, DIM)

    cos, sin, P64, E16 = _rope_tables()
    G0, G1, G2 = _gate_selectors()

    qkv, gates = pl.pallas_call(
        _qkv_kernel,
        grid=(GQ,),
        in_specs=[
            pl.BlockSpec((BQ, DIM), lambda i: (i, 0)),
            pl.BlockSpec((1, DIM), lambda i: (0, 0)),
            pl.BlockSpec((DIM, 3 * inner), lambda i: (0, 0)),
            pl.BlockSpec((DIM, 3 * h), lambda i: (0, 0)),
        ],
        out_specs=[
            pl.BlockSpec((BQ, 3 * inner), lambda i: (i, 0)),
            pl.BlockSpec((BQ, 3 * h), lambda i: (i, 0)),
        ],
        out_shape=[
            jax.ShapeDtypeStruct((n, 3 * inner), jnp.float32),
            jax.ShapeDtypeStruct((n, 3 * h), jnp.float32),
        ],
    )(x2, g_norm.reshape(1, DIM), W_qkv, W_comb)

    q = qkv[:, :inner].reshape(n, h, dh).transpose(1, 0, 2)
    k = qkv[:, inner:2 * inner].reshape(n, h, dh).transpose(1, 0, 2)
    v = qkv[:, 2 * inner:].reshape(n, h, dh).transpose(1, 0, 2)

    rows = h * NBLK
    brows = rows // 2
    kc_in = k.reshape(rows, cdim)
    vc_in = v.reshape(rows, cdim)
    kin_full = jnp.broadcast_to(k_intra.reshape(h, 1, cdim),
                                (h, NBLK, cdim)).reshape(rows, cdim)
    vin_full = jnp.broadcast_to(v_intra.reshape(h, 1, cdim),
                                (h, NBLK, cdim)).reshape(rows, cdim)
    ck2, cv2 = pl.pallas_call(
        _cmlp_kernel,
        grid=(2,),
        in_specs=[
            pl.BlockSpec((brows, cdim), lambda i: (i, 0)),
            pl.BlockSpec((brows, cdim), lambda i: (i, 0)),
            pl.BlockSpec((brows, cdim), lambda i: (i, 0)),
            pl.BlockSpec((brows, cdim), lambda i: (i, 0)),
            pl.BlockSpec((cdim, cdim), lambda i: (0, 0)),
            pl.BlockSpec((1, cdim), lambda i: (0, 0)),
            pl.BlockSpec((cdim, dh), lambda i: (0, 0)),
            pl.BlockSpec((1, dh), lambda i: (0, 0)),
            pl.BlockSpec((cdim, cdim), lambda i: (0, 0)),
            pl.BlockSpec((1, cdim), lambda i: (0, 0)),
            pl.BlockSpec((cdim, dh), lambda i: (0, 0)),
            pl.BlockSpec((1, dh), lambda i: (0, 0)),
        ],
        out_specs=[
            pl.BlockSpec((brows, dh), lambda i: (i, 0)),
            pl.BlockSpec((brows, dh), lambda i: (i, 0)),
        ],
        out_shape=[
            jax.ShapeDtypeStruct((rows, dh), jnp.float32),
            jax.ShapeDtypeStruct((rows, dh), jnp.float32),
        ],
    )(kc_in, vc_in, kin_full, vin_full,
      Wk1, bk1.reshape(1, cdim), Wk2, bk2.reshape(1, dh),
      Wv1, bv1.reshape(1, cdim), Wv2, bv2.reshape(1, dh))
    ck = ck2.reshape(h, NBLK, dh)
    cv = cv2.reshape(h, NBLK, dh)

    ck_full = jnp.concatenate(
        (jnp.broadcast_to(mem_kv[0], (h, NUM_MEM, dh)), ck), axis=1)
    cv_full = jnp.concatenate(
        (jnp.broadcast_to(mem_kv[1], (h, NUM_MEM, dh)), cv), axis=1)

    co, sidx, sval = pl.pallas_call(
        _cattn_kernel,
        grid=(h, GQ),
        in_specs=[
            pl.BlockSpec((1, BQ, dh), lambda i, j: (i, j, 0)),
            pl.BlockSpec((1, CTX, dh), lambda i, j: (i, 0, 0)),
            pl.BlockSpec((1, CTX, dh), lambda i, j: (i, 0, 0)),
        ],
        out_specs=[
            pl.BlockSpec((1, BQ, dh), lambda i, j: (i, j, 0)),
            pl.BlockSpec((1, BQ, NUM_SEL), lambda i, j: (i, j, 0)),
            pl.BlockSpec((1, BQ, NUM_SEL), lambda i, j: (i, j, 0)),
        ],
        out_shape=[
            jax.ShapeDtypeStruct((h, n, dh), jnp.float32),
            jax.ShapeDtypeStruct((h, n, NUM_SEL), jnp.int32),
            jax.ShapeDtypeStruct((h, n, NUM_SEL), jnp.float32),
        ],
    )(q, ck_full, cv_full)

    fo, lo = pl.pallas_call(
        _fs_kernel,
        grid=(h, GQ),
        in_specs=[
            pl.BlockSpec((1, BQ, dh), lambda i, j: (i, j, 0)),
            pl.BlockSpec((1, n, dh), lambda i, j: (i, 0, 0)),
            pl.BlockSpec((1, n, dh), lambda i, j: (i, 0, 0)),
            pl.BlockSpec((n, dh), lambda i, j: (0, 0)),
            pl.BlockSpec((n, dh), lambda i, j: (0, 0)),
            pl.BlockSpec((dh, dh), lambda i, j: (0, 0)),
            pl.BlockSpec((NB_T, BK), lambda i, j: (0, 0)),
            pl.BlockSpec((1, BQ, NUM_SEL), lambda i, j: (i, j, 0)),
            pl.BlockSpec((1, BQ, NUM_SEL), lambda i, j: (i, j, 0)),
        ],
        out_specs=[
            pl.BlockSpec((1, BQ, dh), lambda i, j: (i, j, 0)),
            pl.BlockSpec((1, BQ, dh), lambda i, j: (i, j, 0)),
        ],
        out_shape=[
            jax.ShapeDtypeStruct((h, n, dh), jnp.float32),
            jax.ShapeDtypeStruct((h, n, dh), jnp.float32),
        ],
        scratch_shapes=[
            pltpu.VMEM((n, dh), jnp.float32),
            pltpu.VMEM((n, 2 * dh), jnp.float32),
            pltpu.VMEM((1, 1), jnp.float32),
        ],
    )(q, k, v, cos, sin, P64, E16, sidx, sval)

    co_f = co.transpose(1, 0, 2).reshape(n, inner)
    fo_f = fo.transpose(1, 0, 2).reshape(n, inner)
    lo_f = lo.transpose(1, 0, 2).reshape(n, inner)
    out = pl.pallas_call(
        _comb_kernel,
        grid=(GQ,),
        in_specs=[
            pl.BlockSpec((BQ, 3 * h), lambda i: (i, 0)),
            pl.BlockSpec((1, 3 * h), lambda i: (0, 0)),
            pl.BlockSpec((BQ, inner), lambda i: (i, 0)),
            pl.BlockSpec((BQ, inner), lambda i: (i, 0)),
            pl.BlockSpec((BQ, inner), lambda i: (i, 0)),
            pl.BlockSpec((3 * h, DIM), lambda i: (0, 0)),
            pl.BlockSpec((3 * h, DIM), lambda i: (0, 0)),
            pl.BlockSpec((3 * h, DIM), lambda i: (0, 0)),
            pl.BlockSpec((inner, DIM), lambda i: (0, 0)),
        ],
        out_specs=pl.BlockSpec((BQ, DIM), lambda i: (i, 0)),
        out_shape=jax.ShapeDtypeStruct((n, DIM), jnp.float32),
    )(gates, b_comb.reshape(1, 3 * h), co_f, fo_f, lo_f, G0, G1, G2, W_out)

    return out.reshape(BATCH, n, DIM)


# speedup vs baseline: 1.2093x; 1.2093x over previous
import functools

import jax
import jax.numpy as jnp
import numpy as np
from jax.experimental import pallas as pl
from jax.experimental.pallas import tpu as pltpu

BATCH = 1
SEQ = 2048
DIM = 768
HEADS = 12
DIM_HEAD = 64
SLIDING = 64
CBS = 16
SBS = 16
NUM_SEL = 4
NUM_MEM = 4
SCALE = DIM_HEAD ** -0.5
NBLK = SEQ // CBS
CTX = NUM_MEM + NBLK
NEG = -1e30

BQ = 256
BK = 256
NT = SEQ // BK
GQ = SEQ // BQ


def _rope_tables():
    inv = 1.0 / (10000.0 ** (np.arange(0, DIM_HEAD, 2, dtype=np.float64) / DIM_HEAD))
    f = np.arange(SEQ, dtype=np.float64)[:, None] * inv[None, :]
    f = np.repeat(f, 2, axis=-1)
    cos = np.cos(f.astype(np.float32)).astype(np.float32)
    sin = np.sin(f.astype(np.float32)).astype(np.float32)
    P = np.zeros((DIM_HEAD, DIM_HEAD), np.float32)
    for k in range(DIM_HEAD // 2):
        P[2 * k + 1, 2 * k] = -1.0
        P[2 * k, 2 * k + 1] = 1.0
    E16 = np.zeros((BK // SBS, BK), np.float32)
    for b in range(BK // SBS):
        E16[b, b * SBS:(b + 1) * SBS] = 1.0
    return jnp.asarray(cos), jnp.asarray(sin), jnp.asarray(P), jnp.asarray(E16)


def _gate_selectors():
    sels = []
    for j in range(3):
        G = np.zeros((3 * HEADS, DIM), np.float32)
        for h in range(HEADS):
            G[h * 3 + j, h * DIM_HEAD:(h + 1) * DIM_HEAD] = 1.0
        sels.append(jnp.asarray(G))
    return sels


def _qkv_kernel(x_ref, gn_ref, wqkv_ref, wcomb_ref, qkv_ref, gate_ref):
    x = x_ref[...]
    ms = jnp.mean(x * x, axis=-1, keepdims=True)
    xn = x * jax.lax.rsqrt(ms + jnp.finfo(jnp.float32).eps) * gn_ref[...]
    qkv_ref[...] = jnp.dot(xn, wqkv_ref[...], preferred_element_type=jnp.float32)
    gate_ref[...] = jnp.dot(xn, wcomb_ref[...], preferred_element_type=jnp.float32)


def _cmlp_kernel(kc_ref, vc_ref, kin_ref, vin_ref, wk1_ref, bk1_ref, wk2_ref,
                 bk2_ref, wv1_ref, bv1_ref, wv2_ref, bv2_ref, ck_ref, cv_ref):
    kc = kc_ref[...] + kin_ref[...]
    vc = vc_ref[...] + vin_ref[...]
    h1 = jnp.maximum(jnp.dot(kc, wk1_ref[...], preferred_element_type=jnp.float32) + bk1_ref[...], 0.0)
    ck_ref[...] = jnp.dot(h1, wk2_ref[...], preferred_element_type=jnp.float32) + bk2_ref[...]
    h2 = jnp.maximum(jnp.dot(vc, wv1_ref[...], preferred_element_type=jnp.float32) + bv1_ref[...], 0.0)
    cv_ref[...] = jnp.dot(h2, wv2_ref[...], preferred_element_type=jnp.float32) + bv2_ref[...]


def _cattn_kernel(q_ref, ck_ref, cv_ref, co_ref, sidx_ref, sval_ref):
    g = pl.program_id(1)
    q = q_ref[0]
    ck = ck_ref[0]
    cv = cv_ref[0]
    sim = jax.lax.dot_general(q, ck, (((1,), (1,)), ((), ())),
                              preferred_element_type=jnp.float32) * SCALE
    row = g * BQ + jax.lax.broadcasted_iota(jnp.int32, (BQ, CTX), 0)
    col = jax.lax.broadcasted_iota(jnp.int32, (BQ, CTX), 1)
    ckseq = jnp.where(col < NUM_MEM, -1, (col - NUM_MEM + 1) * CBS - 1)
    sim = jnp.where(ckseq < row, sim, NEG)
    m = jnp.max(sim, axis=-1, keepdims=True)
    e = jnp.exp(sim - m)
    p = e / jnp.sum(e, axis=-1, keepdims=True)
    co_ref[0] = jnp.dot(p, cv, preferred_element_type=jnp.float32)
    work = jnp.where(col >= NUM_MEM, p, -1.0)
    idxs, vals = [], []
    for _ in range(NUM_SEL):
        mval = jnp.max(work, axis=-1, keepdims=True)
        cand = jnp.where(work == mval, col, jnp.int32(1 << 30))
        midx = jnp.min(cand, axis=-1, keepdims=True)
        vals.append(mval)
        idxs.append(midx - NUM_MEM)
        work = jnp.where(col == midx, -1.0, work)
    sidx_ref[0] = jnp.concatenate(idxs, axis=-1)
    sval_ref[0] = jnp.concatenate(vals, axis=-1)


NB_T = BK // SBS


def _fs_kernel(q_ref, k_ref, v_ref, cos_ref, sin_ref, p64_ref, e16_ref,
               sidx_ref, sval_ref, fo_ref, lo_ref, rk_ref, vext_ref, mk_ref):
    g = pl.program_id(1)
    p64 = p64_ref[...]

    @pl.when(g == 0)
    def _():
        kk = k_ref[0]
        rk = kk * cos_ref[...] + jnp.dot(
            kk, p64, preferred_element_type=jnp.float32) * sin_ref[...]
        rk_ref[...] = rk
        vv = v_ref[0]
        vext_ref[:, :DIM_HEAD] = vv.astype(jnp.bfloat16)
        lane = jax.lax.broadcasted_iota(jnp.int32, (SEQ, DIM_HEAD), 1)
        vext_ref[:, DIM_HEAD:] = jnp.where(lane == 0, 1.0, 0.0).astype(jnp.bfloat16)
        mk_ref[...] = jnp.max(
            jnp.sum(rk * rk, axis=-1, keepdims=True), axis=0, keepdims=True)

    qb = q_ref[0]
    cosq = cos_ref[pl.ds(g * BQ, BQ), :]
    sinq = sin_ref[pl.ds(g * BQ, BQ), :]
    rq = (qb * cosq + jnp.dot(qb, p64, preferred_element_type=jnp.float32)
          * sinq) * SCALE
    nq = jnp.sqrt(jnp.sum(rq * rq, axis=-1, keepdims=True))
    m0 = nq * jnp.sqrt(mk_ref[...])

    qpos_r = g * BQ + jax.lax.broadcasted_iota(jnp.int32, (BQ, 1), 0)
    own_w = qpos_r // SBS
    sidx = sidx_ref[0]
    valid = sval_ref[0] > 1e-10

    e16 = e16_ref[...]
    colb = jax.lax.broadcasted_iota(jnp.int32, (BQ, NB_T), 1)
    acc = jnp.zeros((BQ, 2 * DIM_HEAD), jnp.float32)
    for t in range(NT):
        kt = rk_ref[t * BK:(t + 1) * BK, :]
        vt = vext_ref[t * BK:(t + 1) * BK, :]
        s = jax.lax.dot_general(rq, kt, (((1,), (1,)), ((), ())),
                                preferred_element_type=jnp.float32)
        wb = jnp.zeros((BQ, NB_T), jnp.float32)
        jbb = t * NB_T + colb
        for si in range(NUM_SEL):
            wb += ((sidx[:, si:si + 1] == jbb) & valid[:, si:si + 1]).astype(jnp.float32)
        w = jnp.dot(wb, e16, preferred_element_type=jnp.float32)
        pt = (w * jnp.exp(s - m0)).astype(jnp.bfloat16)
        acc = acc + jnp.dot(pt, vt, preferred_element_type=jnp.float32)

    SW = BQ + 2 * SLIDING
    start = pl.multiple_of(jnp.maximum(g * BQ - 2 * SLIDING, 0), 2 * SLIDING)
    ks = rk_ref[pl.ds(start, SW), :]
    vs = vext_ref[pl.ds(start, SW), :]
    bsim = jax.lax.dot_general(rq, ks, (((1,), (1,)), ((), ())),
                               preferred_element_type=jnp.float32)
    kpos2 = start + jax.lax.broadcasted_iota(jnp.int32, (BQ, SW), 1)
    qpos2 = g * BQ + jax.lax.broadcasted_iota(jnp.int32, (BQ, SW), 0)
    causal = kpos2 <= qpos2
    eb = jnp.exp(bsim - m0)
    e_sl = jnp.where(causal & (qpos2 - kpos2 <= SLIDING), eb, 0.0).astype(jnp.bfloat16)
    sl_ext = jnp.dot(e_sl, vs, preferred_element_type=jnp.float32)
    lo_ref[0] = sl_ext[:, :DIM_HEAD] / sl_ext[:, DIM_HEAD:DIM_HEAD + 1]
    e_own = jnp.where(causal & ((kpos2 // SBS) == own_w), eb, 0.0).astype(jnp.bfloat16)
    acc = acc + jnp.dot(e_own, vs, preferred_element_type=jnp.float32)
    fo_ref[0] = acc[:, :DIM_HEAD] / acc[:, DIM_HEAD:DIM_HEAD + 1]


def _comb_kernel(gate_ref, bcomb_ref, co_ref, fo_ref, lo_ref, g0_ref, g1_ref,
                 g2_ref, wout_ref, out_ref):
    sg = jax.nn.sigmoid(gate_ref[...] + bcomb_ref[...])
    o = (jnp.dot(sg, g0_ref[...], preferred_element_type=jnp.float32) * co_ref[...]
         + jnp.dot(sg, g1_ref[...], preferred_element_type=jnp.float32) * fo_ref[...]
         + jnp.dot(sg, g2_ref[...], preferred_element_type=jnp.float32) * lo_ref[...])
    out_ref[...] = jnp.dot(o.astype(jnp.bfloat16), wout_ref[...].astype(jnp.bfloat16),
                           preferred_element_type=jnp.float32)


def kernel(inp, g_norm, W_qkv, mem_kv, k_intra, v_intra, Wk1, bk1, Wk2, bk2,
           Wv1, bv1, Wv2, bv2, W_comb, b_comb, W_out):
    n, h, dh = SEQ, HEADS, DIM_HEAD
    inner = h * dh
    cdim = CBS * dh
    x2 = inp.reshape(n, DIM)

    cos, sin, P64, E16 = _rope_tables()
    G0, G1, G2 = _gate_selectors()

    qkv, gates = pl.pallas_call(
        _qkv_kernel,
        grid=(GQ,),
        in_specs=[
            pl.BlockSpec((BQ, DIM), lambda i: (i, 0)),
            pl.BlockSpec((1, DIM), lambda i: (0, 0)),
            pl.BlockSpec((DIM, 3 * inner), lambda i: (0, 0)),
            pl.BlockSpec((DIM, 3 * h), lambda i: (0, 0)),
        ],
        out_specs=[
            pl.BlockSpec((BQ, 3 * inner), lambda i: (i, 0)),
            pl.BlockSpec((BQ, 3 * h), lambda i: (i, 0)),
        ],
        out_shape=[
            jax.ShapeDtypeStruct((n, 3 * inner), jnp.float32),
            jax.ShapeDtypeStruct((n, 3 * h), jnp.float32),
        ],
    )(x2, g_norm.reshape(1, DIM), W_qkv, W_comb)

    q = qkv[:, :inner].reshape(n, h, dh).transpose(1, 0, 2)
    k = qkv[:, inner:2 * inner].reshape(n, h, dh).transpose(1, 0, 2)
    v = qkv[:, 2 * inner:].reshape(n, h, dh).transpose(1, 0, 2)

    rows = h * NBLK
    brows = rows // 2
    kc_in = k.reshape(rows, cdim)
    vc_in = v.reshape(rows, cdim)
    kin_full = jnp.broadcast_to(k_intra.reshape(h, 1, cdim),
                                (h, NBLK, cdim)).reshape(rows, cdim)
    vin_full = jnp.broadcast_to(v_intra.reshape(h, 1, cdim),
                                (h, NBLK, cdim)).reshape(rows, cdim)
    ck2, cv2 = pl.pallas_call(
        _cmlp_kernel,
        grid=(2,),
        in_specs=[
            pl.BlockSpec((brows, cdim), lambda i: (i, 0)),
            pl.BlockSpec((brows, cdim), lambda i: (i, 0)),
            pl.BlockSpec((brows, cdim), lambda i: (i, 0)),
            pl.BlockSpec((brows, cdim), lambda i: (i, 0)),
            pl.BlockSpec((cdim, cdim), lambda i: (0, 0)),
            pl.BlockSpec((1, cdim), lambda i: (0, 0)),
            pl.BlockSpec((cdim, dh), lambda i: (0, 0)),
            pl.BlockSpec((1, dh), lambda i: (0, 0)),
            pl.BlockSpec((cdim, cdim), lambda i: (0, 0)),
            pl.BlockSpec((1, cdim), lambda i: (0, 0)),
            pl.BlockSpec((cdim, dh), lambda i: (0, 0)),
            pl.BlockSpec((1, dh), lambda i: (0, 0)),
        ],
        out_specs=[
            pl.BlockSpec((brows, dh), lambda i: (i, 0)),
            pl.BlockSpec((brows, dh), lambda i: (i, 0)),
        ],
        out_shape=[
            jax.ShapeDtypeStruct((rows, dh), jnp.float32),
            jax.ShapeDtypeStruct((rows, dh), jnp.float32),
        ],
    )(kc_in, vc_in, kin_full, vin_full,
      Wk1, bk1.reshape(1, cdim), Wk2, bk2.reshape(1, dh),
      Wv1, bv1.reshape(1, cdim), Wv2, bv2.reshape(1, dh))
    ck = ck2.reshape(h, NBLK, dh)
    cv = cv2.reshape(h, NBLK, dh)

    ck_full = jnp.concatenate(
        (jnp.broadcast_to(mem_kv[0], (h, NUM_MEM, dh)), ck), axis=1)
    cv_full = jnp.concatenate(
        (jnp.broadcast_to(mem_kv[1], (h, NUM_MEM, dh)), cv), axis=1)

    co, sidx, sval = pl.pallas_call(
        _cattn_kernel,
        grid=(h, GQ),
        in_specs=[
            pl.BlockSpec((1, BQ, dh), lambda i, j: (i, j, 0)),
            pl.BlockSpec((1, CTX, dh), lambda i, j: (i, 0, 0)),
            pl.BlockSpec((1, CTX, dh), lambda i, j: (i, 0, 0)),
        ],
        out_specs=[
            pl.BlockSpec((1, BQ, dh), lambda i, j: (i, j, 0)),
            pl.BlockSpec((1, BQ, NUM_SEL), lambda i, j: (i, j, 0)),
            pl.BlockSpec((1, BQ, NUM_SEL), lambda i, j: (i, j, 0)),
        ],
        out_shape=[
            jax.ShapeDtypeStruct((h, n, dh), jnp.float32),
            jax.ShapeDtypeStruct((h, n, NUM_SEL), jnp.int32),
            jax.ShapeDtypeStruct((h, n, NUM_SEL), jnp.float32),
        ],
    )(q, ck_full, cv_full)

    fo, lo = pl.pallas_call(
        _fs_kernel,
        grid=(h, GQ),
        in_specs=[
            pl.BlockSpec((1, BQ, dh), lambda i, j: (i, j, 0)),
            pl.BlockSpec((1, n, dh), lambda i, j: (i, 0, 0)),
            pl.BlockSpec((1, n, dh), lambda i, j: (i, 0, 0)),
            pl.BlockSpec((n, dh), lambda i, j: (0, 0)),
            pl.BlockSpec((n, dh), lambda i, j: (0, 0)),
            pl.BlockSpec((dh, dh), lambda i, j: (0, 0)),
            pl.BlockSpec((NB_T, BK), lambda i, j: (0, 0)),
            pl.BlockSpec((1, BQ, NUM_SEL), lambda i, j: (i, j, 0)),
            pl.BlockSpec((1, BQ, NUM_SEL), lambda i, j: (i, j, 0)),
        ],
        out_specs=[
            pl.BlockSpec((1, BQ, dh), lambda i, j: (i, j, 0)),
            pl.BlockSpec((1, BQ, dh), lambda i, j: (i, j, 0)),
        ],
        out_shape=[
            jax.ShapeDtypeStruct((h, n, dh), jnp.float32),
            jax.ShapeDtypeStruct((h, n, dh), jnp.float32),
        ],
        scratch_shapes=[
            pltpu.VMEM((n, dh), jnp.float32),
            pltpu.VMEM((n, 2 * dh), jnp.bfloat16),
            pltpu.VMEM((1, 1), jnp.float32),
        ],
    )(q, k, v, cos, sin, P64, E16, sidx, sval)

    co_f = co.transpose(1, 0, 2).reshape(n, inner)
    fo_f = fo.transpose(1, 0, 2).reshape(n, inner)
    lo_f = lo.transpose(1, 0, 2).reshape(n, inner)
    out = pl.pallas_call(
        _comb_kernel,
        grid=(GQ,),
        in_specs=[
            pl.BlockSpec((BQ, 3 * h), lambda i: (i, 0)),
            pl.BlockSpec((1, 3 * h), lambda i: (0, 0)),
            pl.BlockSpec((BQ, inner), lambda i: (i, 0)),
            pl.BlockSpec((BQ, inner), lambda i: (i, 0)),
            pl.BlockSpec((BQ, inner), lambda i: (i, 0)),
            pl.BlockSpec((3 * h, DIM), lambda i: (0, 0)),
            pl.BlockSpec((3 * h, DIM), lambda i: (0, 0)),
            pl.BlockSpec((3 * h, DIM), lambda i: (0, 0)),
            pl.BlockSpec((inner, DIM), lambda i: (0, 0)),
        ],
        out_specs=pl.BlockSpec((BQ, DIM), lambda i: (i, 0)),
        out_shape=jax.ShapeDtypeStruct((n, DIM), jnp.float32),
    )(gates, b_comb.reshape(1, 3 * h), co_f, fo_f, lo_f, G0, G1, G2, W_out)

    return out.reshape(BATCH, n, DIM)


# speedup vs baseline: 1.3020x; 1.0767x over previous
import functools

import jax
import jax.numpy as jnp
import numpy as np
from jax.experimental import pallas as pl
from jax.experimental.pallas import tpu as pltpu

BATCH = 1
SEQ = 2048
DIM = 768
HEADS = 12
DIM_HEAD = 64
SLIDING = 64
CBS = 16
SBS = 16
NUM_SEL = 4
NUM_MEM = 4
SCALE = DIM_HEAD ** -0.5
NBLK = SEQ // CBS
CTX = NUM_MEM + NBLK
NEG = -1e30

BQ = 256
BK = 256
NT = SEQ // BK
GQ = SEQ // BQ


def _rope_tables():
    inv = 1.0 / (10000.0 ** (np.arange(0, DIM_HEAD, 2, dtype=np.float64) / DIM_HEAD))
    f = np.arange(SEQ, dtype=np.float64)[:, None] * inv[None, :]
    f = np.repeat(f, 2, axis=-1)
    cos = np.cos(f.astype(np.float32)).astype(np.float32)
    sin = np.sin(f.astype(np.float32)).astype(np.float32)
    P = np.zeros((DIM_HEAD, DIM_HEAD), np.float32)
    for k in range(DIM_HEAD // 2):
        P[2 * k + 1, 2 * k] = -1.0
        P[2 * k, 2 * k + 1] = 1.0
    E16 = np.zeros((BK // SBS, BK), np.float32)
    for b in range(BK // SBS):
        E16[b, b * SBS:(b + 1) * SBS] = 1.0
    return jnp.asarray(cos), jnp.asarray(sin), jnp.asarray(P), jnp.asarray(E16)


def _gate_selectors():
    sels = []
    for j in range(3):
        G = np.zeros((3 * HEADS, DIM), np.float32)
        for h in range(HEADS):
            G[h * 3 + j, h * DIM_HEAD:(h + 1) * DIM_HEAD] = 1.0
        sels.append(jnp.asarray(G))
    return sels


def _qkv_kernel(x_ref, gn_ref, wqkv_ref, wcomb_ref, qkv_ref, gate_ref):
    x = x_ref[...]
    ms = jnp.mean(x * x, axis=-1, keepdims=True)
    xn = x * jax.lax.rsqrt(ms + jnp.finfo(jnp.float32).eps) * gn_ref[...]
    qkv_ref[...] = jnp.dot(xn, wqkv_ref[...], preferred_element_type=jnp.float32)
    gate_ref[...] = jnp.dot(xn, wcomb_ref[...], preferred_element_type=jnp.float32)


def _cmlp_kernel(kc_ref, vc_ref, kin_ref, vin_ref, wk1_ref, bk1_ref, wk2_ref,
                 bk2_ref, wv1_ref, bv1_ref, wv2_ref, bv2_ref, ck_ref, cv_ref):
    kc = kc_ref[...] + kin_ref[...]
    vc = vc_ref[...] + vin_ref[...]
    h1 = jnp.maximum(jnp.dot(kc, wk1_ref[...], preferred_element_type=jnp.float32) + bk1_ref[...], 0.0)
    ck_ref[...] = jnp.dot(h1, wk2_ref[...], preferred_element_type=jnp.float32) + bk2_ref[...]
    h2 = jnp.maximum(jnp.dot(vc, wv1_ref[...], preferred_element_type=jnp.float32) + bv1_ref[...], 0.0)
    cv_ref[...] = jnp.dot(h2, wv2_ref[...], preferred_element_type=jnp.float32) + bv2_ref[...]


def _cattn_kernel(q_ref, ck_ref, cv_ref, co_ref, sidx_ref, sval_ref):
    g = pl.program_id(1)
    q = q_ref[0]
    ck = ck_ref[0]
    cv = cv_ref[0]
    sim = jax.lax.dot_general(q, ck, (((1,), (1,)), ((), ())),
                              preferred_element_type=jnp.float32) * SCALE
    row = g * BQ + jax.lax.broadcasted_iota(jnp.int32, (BQ, CTX), 0)
    col = jax.lax.broadcasted_iota(jnp.int32, (BQ, CTX), 1)
    ckseq = jnp.where(col < NUM_MEM, -1, (col - NUM_MEM + 1) * CBS - 1)
    sim = jnp.where(ckseq < row, sim, NEG)
    m = jnp.max(sim, axis=-1, keepdims=True)
    e = jnp.exp(sim - m)
    p = e / jnp.sum(e, axis=-1, keepdims=True)
    co_ref[0] = jnp.dot(p, cv, preferred_element_type=jnp.float32)
    work = jnp.where(col >= NUM_MEM, p, -1.0)
    idxs, vals = [], []
    for _ in range(NUM_SEL):
        mval = jnp.max(work, axis=-1, keepdims=True)
        cand = jnp.where(work == mval, col, jnp.int32(1 << 30))
        midx = jnp.min(cand, axis=-1, keepdims=True)
        vals.append(mval)
        idxs.append(midx - NUM_MEM)
        work = jnp.where(col == midx, -1.0, work)
    sidx_ref[0] = jnp.concatenate(idxs, axis=-1)
    sval_ref[0] = jnp.concatenate(vals, axis=-1)


NB_T = BK // SBS


def _fs_kernel(q_ref, k_ref, v_ref, cos_ref, sin_ref, p64_ref, e16_ref,
               sidx_ref, sval_ref, fo_ref, lo_ref, rk_ref, vext_ref, mk_ref):
    g = pl.program_id(1)
    p64 = p64_ref[...]

    @pl.when(g == 0)
    def _():
        kk = k_ref[0]
        rk = kk * cos_ref[...] + jnp.dot(
            kk, p64, preferred_element_type=jnp.float32) * sin_ref[...]
        rk_ref[...] = rk
        vv = v_ref[0]
        vext_ref[:, :DIM_HEAD] = vv
        lane = jax.lax.broadcasted_iota(jnp.int32, (SEQ, DIM_HEAD), 1)
        vext_ref[:, DIM_HEAD:] = jnp.where(lane == 0, 1.0, 0.0)
        mk_ref[...] = jnp.max(
            jnp.sum(rk * rk, axis=-1, keepdims=True), axis=0, keepdims=True)

    qb = q_ref[0]
    cosq = cos_ref[pl.ds(g * BQ, BQ), :]
    sinq = sin_ref[pl.ds(g * BQ, BQ), :]
    rq = (qb * cosq + jnp.dot(qb, p64, preferred_element_type=jnp.float32)
          * sinq) * SCALE
    nq = jnp.sqrt(jnp.sum(rq * rq, axis=-1, keepdims=True))
    m0 = nq * jnp.sqrt(mk_ref[...])

    qpos_r = g * BQ + jax.lax.broadcasted_iota(jnp.int32, (BQ, 1), 0)
    own_w = qpos_r // SBS
    sidx = sidx_ref[0]
    valid = sval_ref[0] > 1e-10

    e16 = e16_ref[...]
    colb = jax.lax.broadcasted_iota(jnp.int32, (BQ, NB_T), 1)
    acc = jnp.zeros((BQ, 2 * DIM_HEAD), jnp.float32)
    for t in range(NT):
        kt = rk_ref[t * BK:(t + 1) * BK, :]
        vt = vext_ref[t * BK:(t + 1) * BK, :]
        s = jax.lax.dot_general(rq, kt, (((1,), (1,)), ((), ())),
                                preferred_element_type=jnp.float32)
        wb = jnp.zeros((BQ, NB_T), jnp.float32)
        jbb = t * NB_T + colb
        for si in range(NUM_SEL):
            wb += ((sidx[:, si:si + 1] == jbb) & valid[:, si:si + 1]).astype(jnp.float32)
        w = jnp.dot(wb, e16, preferred_element_type=jnp.float32)
        pt = w * jnp.exp(s - m0)
        acc = acc + jnp.dot(pt, vt, preferred_element_type=jnp.float32)

    SW = BQ + 2 * SLIDING
    start = pl.multiple_of(jnp.maximum(g * BQ - 2 * SLIDING, 0), 2 * SLIDING)
    ks = rk_ref[pl.ds(start, SW), :]
    vs = vext_ref[pl.ds(start, SW), :]
    bsim = jax.lax.dot_general(rq, ks, (((1,), (1,)), ((), ())),
                               preferred_element_type=jnp.float32)
    kpos2 = start + jax.lax.broadcasted_iota(jnp.int32, (BQ, SW), 1)
    qpos2 = g * BQ + jax.lax.broadcasted_iota(jnp.int32, (BQ, SW), 0)
    causal = kpos2 <= qpos2
    eb = jnp.exp(bsim - m0)
    e_sl = jnp.where(causal & (qpos2 - kpos2 <= SLIDING), eb, 0.0)
    sl_ext = jnp.dot(e_sl, vs, preferred_element_type=jnp.float32)
    lo_ref[0] = sl_ext[:, :DIM_HEAD] / sl_ext[:, DIM_HEAD:DIM_HEAD + 1]
    e_own = jnp.where(causal & ((kpos2 // SBS) == own_w), eb, 0.0)
    acc = acc + jnp.dot(e_own, vs, preferred_element_type=jnp.float32)
    fo_ref[0] = acc[:, :DIM_HEAD] / acc[:, DIM_HEAD:DIM_HEAD + 1]


def _comb_kernel(gate_ref, bcomb_ref, co_ref, fo_ref, lo_ref, g0_ref, g1_ref,
                 g2_ref, wout_ref, out_ref):
    sg = jax.nn.sigmoid(gate_ref[...] + bcomb_ref[...])
    o = (jnp.dot(sg, g0_ref[...], preferred_element_type=jnp.float32) * co_ref[...]
         + jnp.dot(sg, g1_ref[...], preferred_element_type=jnp.float32) * fo_ref[...]
         + jnp.dot(sg, g2_ref[...], preferred_element_type=jnp.float32) * lo_ref[...])
    out_ref[...] = jnp.dot(o, wout_ref[...], preferred_element_type=jnp.float32)


def kernel(inp, g_norm, W_qkv, mem_kv, k_intra, v_intra, Wk1, bk1, Wk2, bk2,
           Wv1, bv1, Wv2, bv2, W_comb, b_comb, W_out):
    n, h, dh = SEQ, HEADS, DIM_HEAD
    inner = h * dh
    cdim = CBS * dh
    x2 = inp.reshape(n, DIM)

    cos, sin, P64, E16 = _rope_tables()
    G0, G1, G2 = _gate_selectors()

    qkv, gates = pl.pallas_call(
        _qkv_kernel,
        grid=(GQ,),
        in_specs=[
            pl.BlockSpec((BQ, DIM), lambda i: (i, 0)),
            pl.BlockSpec((1, DIM), lambda i: (0, 0)),
            pl.BlockSpec((DIM, 3 * inner), lambda i: (0, 0)),
            pl.BlockSpec((DIM, 3 * h), lambda i: (0, 0)),
        ],
        out_specs=[
            pl.BlockSpec((BQ, 3 * inner), lambda i: (i, 0)),
            pl.BlockSpec((BQ, 3 * h), lambda i: (i, 0)),
        ],
        out_shape=[
            jax.ShapeDtypeStruct((n, 3 * inner), jnp.float32),
            jax.ShapeDtypeStruct((n, 3 * h), jnp.float32),
        ],
    )(x2, g_norm.reshape(1, DIM), W_qkv, W_comb)

    q = qkv[:, :inner].reshape(n, h, dh).transpose(1, 0, 2)
    k = qkv[:, inner:2 * inner].reshape(n, h, dh).transpose(1, 0, 2)
    v = qkv[:, 2 * inner:].reshape(n, h, dh).transpose(1, 0, 2)

    rows = h * NBLK
    brows = rows // 2
    kc_in = k.reshape(rows, cdim)
    vc_in = v.reshape(rows, cdim)
    kin_full = jnp.broadcast_to(k_intra.reshape(h, 1, cdim),
                                (h, NBLK, cdim)).reshape(rows, cdim)
    vin_full = jnp.broadcast_to(v_intra.reshape(h, 1, cdim),
                                (h, NBLK, cdim)).reshape(rows, cdim)
    ck2, cv2 = pl.pallas_call(
        _cmlp_kernel,
        grid=(2,),
        in_specs=[
            pl.BlockSpec((brows, cdim), lambda i: (i, 0)),
            pl.BlockSpec((brows, cdim), lambda i: (i, 0)),
            pl.BlockSpec((brows, cdim), lambda i: (i, 0)),
            pl.BlockSpec((brows, cdim), lambda i: (i, 0)),
            pl.BlockSpec((cdim, cdim), lambda i: (0, 0)),
            pl.BlockSpec((1, cdim), lambda i: (0, 0)),
            pl.BlockSpec((cdim, dh), lambda i: (0, 0)),
            pl.BlockSpec((1, dh), lambda i: (0, 0)),
            pl.BlockSpec((cdim, cdim), lambda i: (0, 0)),
            pl.BlockSpec((1, cdim), lambda i: (0, 0)),
            pl.BlockSpec((cdim, dh), lambda i: (0, 0)),
            pl.BlockSpec((1, dh), lambda i: (0, 0)),
        ],
        out_specs=[
            pl.BlockSpec((brows, dh), lambda i: (i, 0)),
            pl.BlockSpec((brows, dh), lambda i: (i, 0)),
        ],
        out_shape=[
            jax.ShapeDtypeStruct((rows, dh), jnp.float32),
            jax.ShapeDtypeStruct((rows, dh), jnp.float32),
        ],
    )(kc_in, vc_in, kin_full, vin_full,
      Wk1, bk1.reshape(1, cdim), Wk2, bk2.reshape(1, dh),
      Wv1, bv1.reshape(1, cdim), Wv2, bv2.reshape(1, dh))
    ck = ck2.reshape(h, NBLK, dh)
    cv = cv2.reshape(h, NBLK, dh)

    ck_full = jnp.concatenate(
        (jnp.broadcast_to(mem_kv[0], (h, NUM_MEM, dh)), ck), axis=1)
    cv_full = jnp.concatenate(
        (jnp.broadcast_to(mem_kv[1], (h, NUM_MEM, dh)), cv), axis=1)

    co, sidx, sval = pl.pallas_call(
        _cattn_kernel,
        grid=(h, GQ),
        in_specs=[
            pl.BlockSpec((1, BQ, dh), lambda i, j: (i, j, 0)),
            pl.BlockSpec((1, CTX, dh), lambda i, j: (i, 0, 0)),
            pl.BlockSpec((1, CTX, dh), lambda i, j: (i, 0, 0)),
        ],
        out_specs=[
            pl.BlockSpec((1, BQ, dh), lambda i, j: (i, j, 0)),
            pl.BlockSpec((1, BQ, NUM_SEL), lambda i, j: (i, j, 0)),
            pl.BlockSpec((1, BQ, NUM_SEL), lambda i, j: (i, j, 0)),
        ],
        out_shape=[
            jax.ShapeDtypeStruct((h, n, dh), jnp.float32),
            jax.ShapeDtypeStruct((h, n, NUM_SEL), jnp.int32),
            jax.ShapeDtypeStruct((h, n, NUM_SEL), jnp.float32),
        ],
    )(q, ck_full, cv_full)

    fo, lo = pl.pallas_call(
        _fs_kernel,
        grid=(h, GQ),
        in_specs=[
            pl.BlockSpec((1, BQ, dh), lambda i, j: (i, j, 0)),
            pl.BlockSpec((1, n, dh), lambda i, j: (i, 0, 0)),
            pl.BlockSpec((1, n, dh), lambda i, j: (i, 0, 0)),
            pl.BlockSpec((n, dh), lambda i, j: (0, 0)),
            pl.BlockSpec((n, dh), lambda i, j: (0, 0)),
            pl.BlockSpec((dh, dh), lambda i, j: (0, 0)),
            pl.BlockSpec((NB_T, BK), lambda i, j: (0, 0)),
            pl.BlockSpec((1, BQ, NUM_SEL), lambda i, j: (i, j, 0)),
            pl.BlockSpec((1, BQ, NUM_SEL), lambda i, j: (i, j, 0)),
        ],
        out_specs=[
            pl.BlockSpec((1, BQ, dh), lambda i, j: (i, j, 0)),
            pl.BlockSpec((1, BQ, dh), lambda i, j: (i, j, 0)),
        ],
        out_shape=[
            jax.ShapeDtypeStruct((h, n, dh), jnp.float32),
            jax.ShapeDtypeStruct((h, n, dh), jnp.float32),
        ],
        scratch_shapes=[
            pltpu.VMEM((n, dh), jnp.float32),
            pltpu.VMEM((n, 2 * dh), jnp.float32),
            pltpu.VMEM((1, 1), jnp.float32),
        ],
    )(q, k, v, cos, sin, P64, E16, sidx, sval)

    co_f = co.transpose(1, 0, 2).reshape(n, inner)
    fo_f = fo.transpose(1, 0, 2).reshape(n, inner)
    lo_f = lo.transpose(1, 0, 2).reshape(n, inner)
    out = pl.pallas_call(
        _comb_kernel,
        grid=(GQ,),
        in_specs=[
            pl.BlockSpec((BQ, 3 * h), lambda i: (i, 0)),
            pl.BlockSpec((1, 3 * h), lambda i: (0, 0)),
            pl.BlockSpec((BQ, inner), lambda i: (i, 0)),
            pl.BlockSpec((BQ, inner), lambda i: (i, 0)),
            pl.BlockSpec((BQ, inner), lambda i: (i, 0)),
            pl.BlockSpec((3 * h, DIM), lambda i: (0, 0)),
            pl.BlockSpec((3 * h, DIM), lambda i: (0, 0)),
            pl.BlockSpec((3 * h, DIM), lambda i: (0, 0)),
            pl.BlockSpec((inner, DIM), lambda i: (0, 0)),
        ],
        out_specs=pl.BlockSpec((BQ, DIM), lambda i: (i, 0)),
        out_shape=jax.ShapeDtypeStruct((n, DIM), jnp.float32),
    )(gates, b_comb.reshape(1, 3 * h), co_f, fo_f, lo_f, G0, G1, G2, W_out)

    return out.reshape(BATCH, n, DIM)


# speedup vs baseline: 1.6405x; 1.2600x over previous
import jax
import jax.numpy as jnp
import numpy as np
from jax.experimental import pallas as pl
from jax.experimental.pallas import tpu as pltpu

BATCH = 1
SEQ = 2048
DIM = 768
HEADS = 12
DIM_HEAD = 64
SLIDING = 64
CBS = 16
SBS = 16
NUM_SEL = 4
NUM_MEM = 4
SCALE = DIM_HEAD ** -0.5
NBLK = SEQ // CBS
CTX = NUM_MEM + NBLK
NEG = -1e30
INNER = HEADS * DIM_HEAD
CDIM = CBS * DIM_HEAD

BQ = 256
BK = 256
NT = SEQ // BK
GQ = SEQ // BQ
NB_T = BK // SBS
SW = BQ + 2 * SLIDING


def _tables():
    inv = 1.0 / (10000.0 ** (np.arange(0, DIM_HEAD, 2, dtype=np.float64) / DIM_HEAD))
    f = np.arange(SEQ, dtype=np.float64)[:, None] * inv[None, :]
    f = np.repeat(f, 2, axis=-1)
    cos = np.cos(f.astype(np.float32)).astype(np.float32)
    sin = np.sin(f.astype(np.float32)).astype(np.float32)
    P = np.zeros((DIM_HEAD, DIM_HEAD), np.float32)
    for k in range(DIM_HEAD // 2):
        P[2 * k + 1, 2 * k] = -1.0
        P[2 * k, 2 * k + 1] = 1.0
    E16 = np.zeros((NB_T, BK), np.float32)
    for b in range(NB_T):
        E16[b, b * SBS:(b + 1) * SBS] = 1.0
    return jnp.asarray(cos), jnp.asarray(sin), jnp.asarray(P), jnp.asarray(E16)


def _gate_selectors():
    sels = []
    for j in range(3):
        G = np.zeros((3 * HEADS, DIM), np.float32)
        for h in range(HEADS):
            G[h * 3 + j, h * DIM_HEAD:(h + 1) * DIM_HEAD] = 1.0
        sels.append(jnp.asarray(G))
    return sels


def _qkv_kernel(x_ref, gn_ref, wq_ref, wk_ref, wv_ref, wcomb_ref,
                q_ref, k_ref, v_ref, gate_ref):
    x = x_ref[...]
    ms = jnp.mean(x * x, axis=-1, keepdims=True)
    xn = x * jax.lax.rsqrt(ms + jnp.finfo(jnp.float32).eps) * gn_ref[...]
    q_ref[...] = jnp.dot(xn, wq_ref[...], preferred_element_type=jnp.float32)
    k_ref[...] = jnp.dot(xn, wk_ref[...], preferred_element_type=jnp.float32)
    v_ref[...] = jnp.dot(xn, wv_ref[...], preferred_element_type=jnp.float32)
    gate_ref[...] = jnp.dot(xn, wcomb_ref[...], preferred_element_type=jnp.float32)


def _cmlp_kernel(kc_ref, vc_ref, kin_ref, vin_ref, wk1_ref, bk1_ref, wk2_ref,
                 bk2_ref, wv1_ref, bv1_ref, wv2_ref, bv2_ref, ck_ref, cv_ref):
    kc = kc_ref[...] + kin_ref[...]
    vc = vc_ref[...] + vin_ref[...]
    h1 = jnp.maximum(jnp.dot(kc, wk1_ref[...], preferred_element_type=jnp.float32) + bk1_ref[...], 0.0)
    ck_ref[...] = jnp.dot(h1, wk2_ref[...], preferred_element_type=jnp.float32) + bk2_ref[...]
    h2 = jnp.maximum(jnp.dot(vc, wv1_ref[...], preferred_element_type=jnp.float32) + bv1_ref[...], 0.0)
    cv_ref[...] = jnp.dot(h2, wv2_ref[...], preferred_element_type=jnp.float32) + bv2_ref[...]


def _ce_kernel(q_ref, k_ref, v_ref, ck_ref, cv_ref, cos_ref, sin_ref,
               p64_ref, e16_ref, co_ref, fo_ref, lo_ref,
               rk_ref, vext_ref, mk_ref):
    g = pl.program_id(1)
    p64 = p64_ref[...]

    @pl.when(g == 0)
    def _():
        lane = jax.lax.broadcasted_iota(jnp.int32, (SEQ, DIM_HEAD), 1)
        ones_col = jnp.where(lane == 0, 1.0, 0.0)
        for hh in range(2):
            sl = slice(hh * DIM_HEAD, (hh + 1) * DIM_HEAD)
            kk = k_ref[:, sl]
            rk = kk * cos_ref[...] + jnp.dot(
                kk, p64, preferred_element_type=jnp.float32) * sin_ref[...]
            rk_ref[hh] = rk
            vext_ref[hh, :, :DIM_HEAD] = v_ref[:, sl]
            vext_ref[hh, :, DIM_HEAD:] = ones_col
            mk_ref[0:1, hh:hh + 1] = jnp.max(
                jnp.sum(rk * rk, axis=-1, keepdims=True), axis=0, keepdims=True)

    rowc = g * BQ + jax.lax.broadcasted_iota(jnp.int32, (BQ, CTX), 0)
    colc = jax.lax.broadcasted_iota(jnp.int32, (BQ, CTX), 1)
    ckseq = jnp.where(colc < NUM_MEM, -1, (colc - NUM_MEM + 1) * CBS - 1)
    cmask = ckseq < rowc
    cosq = cos_ref[pl.ds(g * BQ, BQ), :]
    sinq = sin_ref[pl.ds(g * BQ, BQ), :]
    qpos_r = g * BQ + jax.lax.broadcasted_iota(jnp.int32, (BQ, 1), 0)
    own_w = qpos_r // SBS
    e16 = e16_ref[...]
    colb = jax.lax.broadcasted_iota(jnp.int32, (BQ, NB_T), 1)
    start = pl.multiple_of(jnp.maximum(g * BQ - 2 * SLIDING, 0), 2 * SLIDING)
    kpos2 = start + jax.lax.broadcasted_iota(jnp.int32, (BQ, SW), 1)
    qpos2 = g * BQ + jax.lax.broadcasted_iota(jnp.int32, (BQ, SW), 0)
    causal = kpos2 <= qpos2
    band = causal & (qpos2 - kpos2 <= SLIDING)

    for hh in range(2):
        sl = slice(hh * DIM_HEAD, (hh + 1) * DIM_HEAD)
        qb = q_ref[:, sl]

        ck = ck_ref[hh]
        cv = cv_ref[hh]
        csim = jax.lax.dot_general(qb, ck, (((1,), (1,)), ((), ())),
                                   preferred_element_type=jnp.float32) * SCALE
        csim = jnp.where(cmask, csim, NEG)
        mC = jnp.max(csim, axis=-1, keepdims=True)
        eC = jnp.exp(csim - mC)
        p = eC / jnp.sum(eC, axis=-1, keepdims=True)
        co_ref[:, sl] = jnp.dot(p, cv, preferred_element_type=jnp.float32)

        penc = (jax.lax.bitcast_convert_type(p, jnp.int32) & jnp.int32(~0xFF)) | (255 - colc)
        work = jnp.where(colc >= NUM_MEM, penc, -1)
        sel_i = []
        sel_ok = []
        for _ in range(NUM_SEL):
            m = jnp.max(work, axis=-1, keepdims=True)
            work = jnp.where(work == m, -1, work)
            sel_i.append(255 - (m & 0xFF) - NUM_MEM)
            vf = jax.lax.bitcast_convert_type(m & jnp.int32(~0xFF), jnp.float32)
            sel_ok.append(vf > 1e-10)

        rq = (qb * cosq + jnp.dot(qb, p64, preferred_element_type=jnp.float32)
              * sinq) * SCALE
        nq = jnp.sqrt(jnp.sum(rq * rq, axis=-1, keepdims=True))
        m0 = nq * jnp.sqrt(mk_ref[0:1, hh:hh + 1])

        acc = jnp.zeros((BQ, 2 * DIM_HEAD), jnp.float32)
        for t in range(NT):
            kt = rk_ref[hh, t * BK:(t + 1) * BK, :]
            vt = vext_ref[hh, t * BK:(t + 1) * BK, :]
            s = jax.lax.dot_general(rq, kt, (((1,), (1,)), ((), ())),
                                    preferred_element_type=jnp.float32)
            wb = jnp.zeros((BQ, NB_T), jnp.float32)
            jbb = t * NB_T + colb
            for si in range(NUM_SEL):
                wb += ((sel_i[si] == jbb) & sel_ok[si]).astype(jnp.float32)
            w = jnp.dot(wb, e16, preferred_element_type=jnp.float32)
            pt = w * jnp.exp(s - m0)
            acc = acc + jnp.dot(pt, vt, preferred_element_type=jnp.float32)

        ks = rk_ref[hh, pl.ds(start, SW), :]
        vs = vext_ref[hh, pl.ds(start, SW), :]
        bsim = jax.lax.dot_general(rq, ks, (((1,), (1,)), ((), ())),
                                   preferred_element_type=jnp.float32)
        eb = jnp.exp(bsim - m0)
        e_sl = jnp.where(band, eb, 0.0)
        sl_ext = jnp.dot(e_sl, vs, preferred_element_type=jnp.float32)
        lo_ref[:, sl] = sl_ext[:, :DIM_HEAD] / sl_ext[:, DIM_HEAD:DIM_HEAD + 1]
        e_own = jnp.where(causal & ((kpos2 // SBS) == own_w), eb, 0.0)
        acc = acc + jnp.dot(e_own, vs, preferred_element_type=jnp.float32)
        fo_ref[:, sl] = acc[:, :DIM_HEAD] / acc[:, DIM_HEAD:DIM_HEAD + 1]


def _comb_kernel(gate_ref, bcomb_ref, co_ref, fo_ref, lo_ref, g0_ref, g1_ref,
                 g2_ref, wout_ref, out_ref):
    sg = jax.nn.sigmoid(gate_ref[...] + bcomb_ref[...])
    o = (jnp.dot(sg, g0_ref[...], preferred_element_type=jnp.float32) * co_ref[...]
         + jnp.dot(sg, g1_ref[...], preferred_element_type=jnp.float32) * fo_ref[...]
         + jnp.dot(sg, g2_ref[...], preferred_element_type=jnp.float32) * lo_ref[...])
    out_ref[...] = jnp.dot(o, wout_ref[...], preferred_element_type=jnp.float32)


def kernel(inp, g_norm, W_qkv, mem_kv, k_intra, v_intra, Wk1, bk1, Wk2, bk2,
           Wv1, bv1, Wv2, bv2, W_comb, b_comb, W_out):
    n, h, dh = SEQ, HEADS, DIM_HEAD
    x2 = inp.reshape(n, DIM)

    cos, sin, P64, E16 = _tables()
    G0, G1, G2 = _gate_selectors()

    q768, k768, v768, gates = pl.pallas_call(
        _qkv_kernel,
        grid=(GQ,),
        in_specs=[
            pl.BlockSpec((BQ, DIM), lambda i: (i, 0)),
            pl.BlockSpec((1, DIM), lambda i: (0, 0)),
            pl.BlockSpec((DIM, INNER), lambda i: (0, 0)),
            pl.BlockSpec((DIM, INNER), lambda i: (0, 0)),
            pl.BlockSpec((DIM, INNER), lambda i: (0, 0)),
            pl.BlockSpec((DIM, 3 * h), lambda i: (0, 0)),
        ],
        out_specs=[
            pl.BlockSpec((BQ, INNER), lambda i: (i, 0)),
            pl.BlockSpec((BQ, INNER), lambda i: (i, 0)),
            pl.BlockSpec((BQ, INNER), lambda i: (i, 0)),
            pl.BlockSpec((BQ, 3 * h), lambda i: (i, 0)),
        ],
        out_shape=[
            jax.ShapeDtypeStruct((n, INNER), jnp.float32),
            jax.ShapeDtypeStruct((n, INNER), jnp.float32),
            jax.ShapeDtypeStruct((n, INNER), jnp.float32),
            jax.ShapeDtypeStruct((n, 3 * h), jnp.float32),
        ],
    )(x2, g_norm.reshape(1, DIM), W_qkv[:, :INNER], W_qkv[:, INNER:2 * INNER],
      W_qkv[:, 2 * INNER:], W_comb)

    rows = h * NBLK
    brows = rows // 2
    kc_in = k768.reshape(NBLK, CBS, h, dh).transpose(2, 0, 1, 3).reshape(rows, CDIM)
    vc_in = v768.reshape(NBLK, CBS, h, dh).transpose(2, 0, 1, 3).reshape(rows, CDIM)
    kin_full = jnp.broadcast_to(k_intra.reshape(h, 1, CDIM),
                                (h, NBLK, CDIM)).reshape(rows, CDIM)
    vin_full = jnp.broadcast_to(v_intra.reshape(h, 1, CDIM),
                                (h, NBLK, CDIM)).reshape(rows, CDIM)
    ck2, cv2 = pl.pallas_call(
        _cmlp_kernel,
        grid=(2,),
        in_specs=[
            pl.BlockSpec((brows, CDIM), lambda i: (i, 0)),
            pl.BlockSpec((brows, CDIM), lambda i: (i, 0)),
            pl.BlockSpec((brows, CDIM), lambda i: (i, 0)),
            pl.BlockSpec((brows, CDIM), lambda i: (i, 0)),
            pl.BlockSpec((CDIM, CDIM), lambda i: (0, 0)),
            pl.BlockSpec((1, CDIM), lambda i: (0, 0)),
            pl.BlockSpec((CDIM, dh), lambda i: (0, 0)),
            pl.BlockSpec((1, dh), lambda i: (0, 0)),
            pl.BlockSpec((CDIM, CDIM), lambda i: (0, 0)),
            pl.BlockSpec((1, CDIM), lambda i: (0, 0)),
            pl.BlockSpec((CDIM, dh), lambda i: (0, 0)),
            pl.BlockSpec((1, dh), lambda i: (0, 0)),
        ],
        out_specs=[
            pl.BlockSpec((brows, dh), lambda i: (i, 0)),
            pl.BlockSpec((brows, dh), lambda i: (i, 0)),
        ],
        out_shape=[
            jax.ShapeDtypeStruct((rows, dh), jnp.float32),
            jax.ShapeDtypeStruct((rows, dh), jnp.float32),
        ],
    )(kc_in, vc_in, kin_full, vin_full,
      Wk1, bk1.reshape(1, CDIM), Wk2, bk2.reshape(1, dh),
      Wv1, bv1.reshape(1, CDIM), Wv2, bv2.reshape(1, dh))

    ck_full = jnp.concatenate(
        (jnp.broadcast_to(mem_kv[0], (h, NUM_MEM, dh)), ck2.reshape(h, NBLK, dh)),
        axis=1)
    cv_full = jnp.concatenate(
        (jnp.broadcast_to(mem_kv[1], (h, NUM_MEM, dh)), cv2.reshape(h, NBLK, dh)),
        axis=1)

    co, fo, lo = pl.pallas_call(
        _ce_kernel,
        grid=(h // 2, GQ),
        in_specs=[
            pl.BlockSpec((BQ, 2 * dh), lambda i, j: (j, i)),
            pl.BlockSpec((n, 2 * dh), lambda i, j: (0, i)),
            pl.BlockSpec((n, 2 * dh), lambda i, j: (0, i)),
            pl.BlockSpec((2, CTX, dh), lambda i, j: (i, 0, 0)),
            pl.BlockSpec((2, CTX, dh), lambda i, j: (i, 0, 0)),
            pl.BlockSpec((n, dh), lambda i, j: (0, 0)),
            pl.BlockSpec((n, dh), lambda i, j: (0, 0)),
            pl.BlockSpec((dh, dh), lambda i, j: (0, 0)),
            pl.BlockSpec((NB_T, BK), lambda i, j: (0, 0)),
        ],
        out_specs=[
            pl.BlockSpec((BQ, 2 * dh), lambda i, j: (j, i)),
            pl.BlockSpec((BQ, 2 * dh), lambda i, j: (j, i)),
            pl.BlockSpec((BQ, 2 * dh), lambda i, j: (j, i)),
        ],
        out_shape=[
            jax.ShapeDtypeStruct((n, INNER), jnp.float32),
            jax.ShapeDtypeStruct((n, INNER), jnp.float32),
            jax.ShapeDtypeStruct((n, INNER), jnp.float32),
        ],
        scratch_shapes=[
            pltpu.VMEM((2, n, dh), jnp.float32),
            pltpu.VMEM((2, n, 2 * dh), jnp.float32),
            pltpu.VMEM((1, 2), jnp.float32),
        ],
    )(q768, k768, v768, ck_full, cv_full, cos, sin, P64, E16)

    out = pl.pallas_call(
        _comb_kernel,
        grid=(GQ,),
        in_specs=[
            pl.BlockSpec((BQ, 3 * h), lambda i: (i, 0)),
            pl.BlockSpec((1, 3 * h), lambda i: (0, 0)),
            pl.BlockSpec((BQ, INNER), lambda i: (i, 0)),
            pl.BlockSpec((BQ, INNER), lambda i: (i, 0)),
            pl.BlockSpec((BQ, INNER), lambda i: (i, 0)),
            pl.BlockSpec((3 * h, DIM), lambda i: (0, 0)),
            pl.BlockSpec((3 * h, DIM), lambda i: (0, 0)),
            pl.BlockSpec((3 * h, DIM), lambda i: (0, 0)),
            pl.BlockSpec((INNER, DIM), lambda i: (0, 0)),
        ],
        out_specs=pl.BlockSpec((BQ, DIM), lambda i: (i, 0)),
        out_shape=jax.ShapeDtypeStruct((n, DIM), jnp.float32),
    )(gates, b_comb.reshape(1, 3 * h), co, fo, lo, G0, G1, G2, W_out)

    return out.reshape(BATCH, n, DIM)
